# baseline (speedup 1.0000x reference)
import functools

import jax
import jax.numpy as jnp
from jax import lax
from jax.experimental import pallas as pl
from jax.experimental.pallas import tpu as pltpu
from jax.experimental.pallas import tpu_sc as plsc

N = 100000
D = 128
CHUNK = 400
TILES = N // CHUNK
NW = 32
ROUNDS = (TILES + NW - 1) // NW


def _body(pt_hbm, eta_hbm, phi_hbm, cls_hbm, tab_hbm, out_hbm,
          idx_v, pt_v, eta_v, phi_v, rows_v, sem):
    wid = lax.axis_index("s") * 2 + lax.axis_index("c")

    lanes = lax.iota(jnp.int32, 16)
    col0 = jnp.zeros((16,), jnp.int32)
    col1 = jnp.full((16,), 1, jnp.int32)
    col2 = jnp.full((16,), 2, jnp.int32)

    def round_fn(i, carry):
        t = wid + i * NW

        @pl.when(t < TILES)
        def _():
            base = pl.multiple_of(t * CHUNK, CHUNK)
            pltpu.sync_copy(cls_hbm.at[pl.ds(base, CHUNK)], idx_v)
            gather = pltpu.async_copy(tab_hbm.at[idx_v], rows_v, sem)
            pltpu.sync_copy(pt_hbm.at[pl.ds(base, CHUNK)], pt_v)
            pltpu.sync_copy(eta_hbm.at[pl.ds(base, CHUNK)], eta_v)
            pltpu.sync_copy(phi_hbm.at[pl.ds(base, CHUNK)], phi_v)
            gather.wait()
            for j in range(CHUNK // 16):
                rows16 = lanes + (j * 16)
                plsc.store_scatter(rows_v, [rows16, col0], pt_v[pl.ds(j * 16, 16)])
                plsc.store_scatter(rows_v, [rows16, col1], eta_v[pl.ds(j * 16, 16)])
                plsc.store_scatter(rows_v, [rows16, col2], phi_v[pl.ds(j * 16, 16)])
            pltpu.sync_copy(rows_v, out_hbm.at[pl.ds(base, CHUNK)])

        return carry

    lax.fori_loop(0, ROUNDS, round_fn, 0)


@jax.jit
def kernel(pt, eta, phi, cls, class_embedding):
    tab = jnp.pad(class_embedding, ((0, 0), (3, 0)))
    mesh = plsc.VectorSubcoreMesh(core_axis_name="c", subcore_axis_name="s",
                                  num_cores=2, num_subcores=16)
    run = pl.kernel(
        _body,
        out_type=jax.ShapeDtypeStruct((N, D), jnp.float32),
        mesh=mesh,
        scratch_types=[
            pltpu.VMEM((CHUNK,), jnp.int32),
            pltpu.VMEM((CHUNK,), jnp.float32),
            pltpu.VMEM((CHUNK,), jnp.float32),
            pltpu.VMEM((CHUNK,), jnp.float32),
            pltpu.VMEM((CHUNK, D), jnp.float32),
            pltpu.SemaphoreType.DMA,
        ],
        compiler_params=pltpu.CompilerParams(needs_layout_passes=False),
    )
    return run(pt, eta, phi, cls, tab)


# speedup vs baseline: 13.8551x; 13.8551x over previous
import functools

import jax
import jax.numpy as jnp
from jax import lax
from jax.experimental import pallas as pl
from jax.experimental.pallas import tpu as pltpu
from jax.experimental.pallas import tpu_sc as plsc

N = 100000
D = 128
CHUNK = 400
TILES = N // CHUNK
NW = 32
ROUNDS = (TILES + NW - 1) // NW


def _body(pt_hbm, eta_hbm, phi_hbm, cls_hbm, tab_hbm, out_hbm,
          idx_v, pt_v, eta_v, phi_v, rows_v, tab_v, sem):
    sid = lax.axis_index("s")
    wid = sid * 2 + lax.axis_index("c")

    @pl.when(sid == 0)
    def _():
        pltpu.sync_copy(tab_hbm, tab_v)

    plsc.subcore_barrier()

    lanes = lax.iota(jnp.int32, 16)
    col0 = jnp.zeros((16,), jnp.int32)
    col1 = jnp.full((16,), 1, jnp.int32)
    col2 = jnp.full((16,), 2, jnp.int32)

    def round_fn(i, carry):
        t = wid + i * NW

        @pl.when(t < TILES)
        def _():
            base = pl.multiple_of(t * CHUNK, CHUNK)
            pltpu.sync_copy(cls_hbm.at[pl.ds(base, CHUNK)], idx_v)
            gather = pltpu.async_copy(tab_v.at[idx_v], rows_v, sem)
            pltpu.sync_copy(pt_hbm.at[pl.ds(base, CHUNK)], pt_v)
            pltpu.sync_copy(eta_hbm.at[pl.ds(base, CHUNK)], eta_v)
            pltpu.sync_copy(phi_hbm.at[pl.ds(base, CHUNK)], phi_v)
            gather.wait()
            for j in range(CHUNK // 16):
                rows16 = lanes + (j * 16)
                plsc.store_scatter(rows_v, [rows16, col0], pt_v[pl.ds(j * 16, 16)])
                plsc.store_scatter(rows_v, [rows16, col1], eta_v[pl.ds(j * 16, 16)])
                plsc.store_scatter(rows_v, [rows16, col2], phi_v[pl.ds(j * 16, 16)])
            pltpu.sync_copy(rows_v, out_hbm.at[pl.ds(base, CHUNK)])

        return carry

    lax.fori_loop(0, ROUNDS, round_fn, 0)


@jax.jit
def kernel(pt, eta, phi, cls, class_embedding):
    tab = jnp.pad(class_embedding, ((0, 0), (3, 0)))
    mesh = plsc.VectorSubcoreMesh(core_axis_name="c", subcore_axis_name="s",
                                  num_cores=2, num_subcores=16)
    run = pl.kernel(
        _body,
        out_type=jax.ShapeDtypeStruct((N, D), jnp.float32),
        mesh=mesh,
        scratch_types=[
            pltpu.VMEM((CHUNK,), jnp.int32),
            pltpu.VMEM((CHUNK,), jnp.float32),
            pltpu.VMEM((CHUNK,), jnp.float32),
            pltpu.VMEM((CHUNK,), jnp.float32),
            pltpu.VMEM((CHUNK, D), jnp.float32),
            pltpu.VMEM_SHARED((5, D), jnp.float32),
            pltpu.SemaphoreType.DMA,
        ],
        compiler_params=pltpu.CompilerParams(needs_layout_passes=False),
    )
    return run(pt, eta, phi, cls, tab)


# speedup vs baseline: 14.7035x; 1.0612x over previous
import functools

import jax
import jax.numpy as jnp
from jax import lax
from jax.experimental import pallas as pl
from jax.experimental.pallas import tpu as pltpu
from jax.experimental.pallas import tpu_sc as plsc

N = 100000
D = 128
CHUNK = 400
TILES = N // CHUNK
NW = 32
ROUNDS = (TILES + NW - 1) // NW


def _body(pt_hbm, eta_hbm, phi_hbm, cls_hbm, tab_hbm, out_hbm,
          idx_v, pt_v, eta_v, phi_v, rows_v, tab_v,
          sem_idx, sem_in, sem_g, sem_out):
    sid = lax.axis_index("s")
    wid = sid * 2 + lax.axis_index("c")

    @pl.when(sid == 0)
    def _():
        pltpu.sync_copy(tab_hbm, tab_v)

    plsc.subcore_barrier()

    lanes = lax.iota(jnp.int32, 16)
    col0 = jnp.zeros((16,), jnp.int32)
    col1 = jnp.full((16,), 1, jnp.int32)
    col2 = jnp.full((16,), 2, jnp.int32)

    def tile_of(r):
        return wid + r * NW

    def issue_inputs(r):
        b = r & 1
        t = jnp.minimum(tile_of(r), TILES - 1)
        pltpu.async_copy(cls_hbm.at[t], idx_v[b], sem_idx[b])
        pltpu.async_copy(pt_hbm.at[t], pt_v[b], sem_in[b])
        pltpu.async_copy(eta_hbm.at[t], eta_v[b], sem_in[b])
        pltpu.async_copy(phi_hbm.at[t], phi_v[b], sem_in[b])

    def wait_idx(b):
        pltpu.make_async_copy(cls_hbm.at[0], idx_v[b], sem_idx[b]).wait()

    def wait_pqr(b):
        pltpu.make_async_copy(pt_hbm.at[0], pt_v[b], sem_in[b]).wait()
        pltpu.make_async_copy(eta_hbm.at[0], eta_v[b], sem_in[b]).wait()
        pltpu.make_async_copy(phi_hbm.at[0], phi_v[b], sem_in[b]).wait()

    def wait_out(b):
        pltpu.make_async_copy(rows_v[b], out_hbm.at[0], sem_out[b]).wait()

    for r in range(ROUNDS):
        b = r & 1
        guard = tile_of(r) < TILES

        @pl.when(guard)
        def _(r=r, b=b):
            if r == 0:
                issue_inputs(0)
            if r >= 2:
                wait_out(b)
            wait_idx(b)
            gather = pltpu.async_copy(tab_v.at[idx_v[b]], rows_v[b], sem_g[b])
            if r + 1 < ROUNDS:
                issue_inputs(r + 1)
            wait_pqr(b)
            gather.wait()
            for j in range(CHUNK // 16):
                rows16 = lanes + (j * 16)
                plsc.store_scatter(rows_v[b], [rows16, col0], pt_v[b][pl.ds(j * 16, 16)])
                plsc.store_scatter(rows_v[b], [rows16, col1], eta_v[b][pl.ds(j * 16, 16)])
                plsc.store_scatter(rows_v[b], [rows16, col2], phi_v[b][pl.ds(j * 16, 16)])
            pltpu.async_copy(rows_v[b], out_hbm.at[tile_of(r)], sem_out[b])

    for r in range(1, ROUNDS):
        @pl.when((tile_of(r - 1) < TILES) & (tile_of(r) >= TILES))
        def _(r=r):
            wait_idx(r & 1)
            wait_pqr(r & 1)

    for r in (ROUNDS - 2, ROUNDS - 1):
        @pl.when(tile_of(r) < TILES)
        def _(r=r):
            wait_out(r & 1)


@jax.jit
def kernel(pt, eta, phi, cls, class_embedding):
    tab = jnp.pad(class_embedding, ((0, 0), (3, 0)))
    mesh = plsc.VectorSubcoreMesh(core_axis_name="c", subcore_axis_name="s",
                                  num_cores=2, num_subcores=16)
    run = pl.kernel(
        _body,
        out_type=jax.ShapeDtypeStruct((TILES, CHUNK, D), jnp.float32),
        mesh=mesh,
        scratch_types=[
            [pltpu.VMEM((CHUNK,), jnp.int32)] * 2,
            [pltpu.VMEM((CHUNK,), jnp.float32)] * 2,
            [pltpu.VMEM((CHUNK,), jnp.float32)] * 2,
            [pltpu.VMEM((CHUNK,), jnp.float32)] * 2,
            [pltpu.VMEM((CHUNK, D), jnp.float32)] * 2,
            pltpu.VMEM_SHARED((5, D), jnp.float32),
            [pltpu.SemaphoreType.DMA] * 2,
            [pltpu.SemaphoreType.DMA] * 2,
            [pltpu.SemaphoreType.DMA] * 2,
            [pltpu.SemaphoreType.DMA] * 2,
        ],
        compiler_params=pltpu.CompilerParams(needs_layout_passes=False),
    )
    out = run(pt.reshape(TILES, CHUNK), eta.reshape(TILES, CHUNK),
              phi.reshape(TILES, CHUNK), cls.reshape(TILES, CHUNK), tab)
    return out.reshape(N, D)


# speedup vs baseline: 15.1172x; 1.0281x over previous
import functools

import jax
import jax.numpy as jnp
from jax import lax
from jax.experimental import pallas as pl
from jax.experimental.pallas import tpu as pltpu
from jax.experimental.pallas import tpu_sc as plsc

N = 100000
D = 128
NUM_CLASSES = 5
EMB_DIM = 125
CHUNK = 400
TILES = N // CHUNK
NW = 32
ROUNDS = (TILES + NW - 1) // NW


def _body(pt_hbm, eta_hbm, phi_hbm, cls_hbm, tab_hbm, out_hbm,
          idx_v, pt_v, eta_v, phi_v, rows_v, tab125_v, tab_tmp, tab_v,
          sem_idx, sem_in, sem_g, sem_out):
    sid = lax.axis_index("s")
    wid = sid * 2 + lax.axis_index("c")

    @pl.when(sid == 0)
    def _():
        pltpu.sync_copy(tab_hbm, tab125_v)
        zeros16 = jnp.zeros((16,), jnp.float32)
        for row in range(NUM_CLASSES):
            tab_tmp[row, pl.ds(0, 16)] = zeros16
            for j in range(8):
                v = tab125_v[row, pl.ds(min(16 * j, EMB_DIM - 16), 16)]
                tab_tmp[row, pl.ds(3 + min(16 * j, EMB_DIM - 16), 16)] = v
        pltpu.sync_copy(tab_tmp, tab_v)

    plsc.subcore_barrier()

    lanes = lax.iota(jnp.int32, 16)
    col0 = jnp.zeros((16,), jnp.int32)
    col1 = jnp.full((16,), 1, jnp.int32)
    col2 = jnp.full((16,), 2, jnp.int32)

    def tile_of(r):
        return wid + r * NW

    def issue_inputs(r):
        b = r & 1
        t = jnp.minimum(tile_of(r), TILES - 1)
        pltpu.async_copy(cls_hbm.at[t], idx_v[b], sem_idx[b])
        pltpu.async_copy(pt_hbm.at[t], pt_v[b], sem_in[b])
        pltpu.async_copy(eta_hbm.at[t], eta_v[b], sem_in[b])
        pltpu.async_copy(phi_hbm.at[t], phi_v[b], sem_in[b])

    def wait_idx(b):
        pltpu.make_async_copy(cls_hbm.at[0], idx_v[b], sem_idx[b]).wait()

    def wait_pqr(b):
        pltpu.make_async_copy(pt_hbm.at[0], pt_v[b], sem_in[b]).wait()
        pltpu.make_async_copy(eta_hbm.at[0], eta_v[b], sem_in[b]).wait()
        pltpu.make_async_copy(phi_hbm.at[0], phi_v[b], sem_in[b]).wait()

    def wait_out(b):
        pltpu.make_async_copy(rows_v[b], out_hbm.at[0], sem_out[b]).wait()

    for r in range(ROUNDS):
        b = r & 1
        guard = tile_of(r) < TILES

        @pl.when(guard)
        def _(r=r, b=b):
            if r == 0:
                issue_inputs(0)
            if r >= 2:
                wait_out(b)
            wait_idx(b)
            gather = pltpu.async_copy(tab_v.at[idx_v[b]], rows_v[b], sem_g[b])
            if r + 1 < ROUNDS:
                issue_inputs(r + 1)
            wait_pqr(b)
            gather.wait()
            for j in range(CHUNK // 16):
                rows16 = lanes + (j * 16)
                plsc.store_scatter(rows_v[b], [rows16, col0], pt_v[b][pl.ds(j * 16, 16)])
                plsc.store_scatter(rows_v[b], [rows16, col1], eta_v[b][pl.ds(j * 16, 16)])
                plsc.store_scatter(rows_v[b], [rows16, col2], phi_v[b][pl.ds(j * 16, 16)])
            pltpu.async_copy(rows_v[b], out_hbm.at[tile_of(r)], sem_out[b])

    for r in range(1, ROUNDS):
        @pl.when((tile_of(r - 1) < TILES) & (tile_of(r) >= TILES))
        def _(r=r):
            wait_idx(r & 1)
            wait_pqr(r & 1)

    for r in (ROUNDS - 2, ROUNDS - 1):
        @pl.when(tile_of(r) < TILES)
        def _(r=r):
            wait_out(r & 1)


@jax.jit
def kernel(pt, eta, phi, cls, class_embedding):
    mesh = plsc.VectorSubcoreMesh(core_axis_name="c", subcore_axis_name="s",
                                  num_cores=2, num_subcores=16)
    run = pl.kernel(
        _body,
        out_type=jax.ShapeDtypeStruct((TILES, CHUNK, D), jnp.float32),
        mesh=mesh,
        scratch_types=[
            [pltpu.VMEM((CHUNK,), jnp.int32)] * 2,
            [pltpu.VMEM((CHUNK,), jnp.float32)] * 2,
            [pltpu.VMEM((CHUNK,), jnp.float32)] * 2,
            [pltpu.VMEM((CHUNK,), jnp.float32)] * 2,
            [pltpu.VMEM((CHUNK, D), jnp.float32)] * 2,
            pltpu.VMEM((NUM_CLASSES, EMB_DIM), jnp.float32),
            pltpu.VMEM((NUM_CLASSES, D), jnp.float32),
            pltpu.VMEM_SHARED((NUM_CLASSES, D), jnp.float32),
            [pltpu.SemaphoreType.DMA] * 2,
            [pltpu.SemaphoreType.DMA] * 2,
            [pltpu.SemaphoreType.DMA] * 2,
            [pltpu.SemaphoreType.DMA] * 2,
        ],
        compiler_params=pltpu.CompilerParams(needs_layout_passes=False),
    )
    out = run(pt.reshape(TILES, CHUNK), eta.reshape(TILES, CHUNK),
              phi.reshape(TILES, CHUNK), cls.reshape(TILES, CHUNK),
              class_embedding)
    return out.reshape(N, D)


# speedup vs baseline: 15.5803x; 1.0306x over previous
import jax
import jax.numpy as jnp
from jax import lax
from jax.experimental import pallas as pl
from jax.experimental.pallas import tpu as pltpu
from jax.experimental.pallas import tpu_sc as plsc

N = 100000
D = 128
NUM_CLASSES = 5
EMB_DIM = 125
AW = 25
ROWS_W = N // AW
CHUNK = 400
ROUNDS = ROWS_W // CHUNK
TILES = N // CHUNK


def _body(pt_hbm, eta_hbm, phi_hbm, cls_hbm, tab_hbm, out_hbm,
          idx_v, pt_v, eta_v, phi_v, rows_v, tab125_v, tab_tmp, tab_v,
          sem_g, sem_out):
    sid = lax.axis_index("s")
    wid = sid * 2 + lax.axis_index("c")

    @pl.when(sid == 0)
    def _():
        pltpu.sync_copy(tab_hbm, tab125_v)
        zeros16 = jnp.zeros((16,), jnp.float32)
        for row in range(NUM_CLASSES):
            tab_tmp[row, pl.ds(0, 16)] = zeros16
            for j in range(8):
                src0 = min(16 * j, EMB_DIM - 16)
                tab_tmp[row, pl.ds(3 + src0, 16)] = tab125_v[row, pl.ds(src0, 16)]
        pltpu.sync_copy(tab_tmp, tab_v)

    plsc.subcore_barrier()

    lanes = lax.iota(jnp.int32, 16)
    col0 = jnp.zeros((16,), jnp.int32)
    col1 = jnp.full((16,), 1, jnp.int32)
    col2 = jnp.full((16,), 2, jnp.int32)

    @pl.when(wid < AW)
    def _():
        base = wid * ROWS_W
        pltpu.sync_copy(cls_hbm.at[pl.ds(base, ROWS_W)], idx_v)
        pltpu.sync_copy(pt_hbm.at[pl.ds(base, ROWS_W)], pt_v)
        pltpu.sync_copy(eta_hbm.at[pl.ds(base, ROWS_W)], eta_v)
        pltpu.sync_copy(phi_hbm.at[pl.ds(base, ROWS_W)], phi_v)

        def start_gather(r):
            b = r & 1
            return pltpu.async_copy(
                tab_v.at[idx_v.at[pl.ds(r * CHUNK, CHUNK)]], rows_v[b], sem_g[b])

        def wait_out(b):
            pltpu.make_async_copy(rows_v[b], out_hbm.at[0], sem_out[b]).wait()

        gathers = [None] * ROUNDS
        gathers[0] = start_gather(0)
        tile0 = wid * ROUNDS
        for r in range(ROUNDS):
            b = r & 1
            gathers[r].wait()
            if r + 1 < ROUNDS:
                if r >= 1:
                    wait_out(1 - b)
                gathers[r + 1] = start_gather(r + 1)
            off = r * CHUNK
            for j in range(CHUNK // 16):
                rows16 = lanes + (j * 16)
                plsc.store_scatter(rows_v[b], [rows16, col0], pt_v[pl.ds(off + j * 16, 16)])
                plsc.store_scatter(rows_v[b], [rows16, col1], eta_v[pl.ds(off + j * 16, 16)])
                plsc.store_scatter(rows_v[b], [rows16, col2], phi_v[pl.ds(off + j * 16, 16)])
            pltpu.async_copy(rows_v[b], out_hbm.at[tile0 + r], sem_out[b])

        wait_out(ROUNDS & 1)
        wait_out(1 - (ROUNDS & 1))


@jax.jit
def kernel(pt, eta, phi, cls, class_embedding):
    mesh = plsc.VectorSubcoreMesh(core_axis_name="c", subcore_axis_name="s",
                                  num_cores=2, num_subcores=16)
    run = pl.kernel(
        _body,
        out_type=jax.ShapeDtypeStruct((TILES, CHUNK, D), jnp.float32),
        mesh=mesh,
        scratch_types=[
            pltpu.VMEM((ROWS_W,), jnp.int32),
            pltpu.VMEM((ROWS_W,), jnp.float32),
            pltpu.VMEM((ROWS_W,), jnp.float32),
            pltpu.VMEM((ROWS_W,), jnp.float32),
            [pltpu.VMEM((CHUNK, D), jnp.float32)] * 2,
            pltpu.VMEM((NUM_CLASSES, EMB_DIM), jnp.float32),
            pltpu.VMEM((NUM_CLASSES, D), jnp.float32),
            pltpu.VMEM_SHARED((NUM_CLASSES, D), jnp.float32),
            [pltpu.SemaphoreType.DMA] * 2,
            [pltpu.SemaphoreType.DMA] * 2,
        ],
        compiler_params=pltpu.CompilerParams(needs_layout_passes=False),
    )
    return run(pt, eta, phi, cls, class_embedding).reshape(N, D)
